# Initial kernel scaffold; baseline (speedup 1.0000x reference)
import functools

import jax
import jax.numpy as jnp
from jax import lax
from jax.experimental import pallas as pl
from jax.experimental.pallas import tpu as pltpu
from jax.experimental.pallas import tpu_sc as plsc

F32 = jnp.float32
I32 = jnp.int32

NS = 16
NC = 2
NW = NS * NC
CH = 512
CHROWS = CH // 128


def _node_stage(rt):
    def body(th_ref, c_ref, s_ref):
        th = th_ref[...]
        c_ref[...] = jnp.cos(th)
        s_ref[...] = jnp.sin(th)

    return pl.pallas_call(
        body,
        out_shape=(jax.ShapeDtypeStruct((rt, 128), F32),
                   jax.ShapeDtypeStruct((rt, 128), F32)),
    )


def _finalize_stage(rt):
    def body(c_ref, s_ref, a0m_ref, a1m_ref, a0c_ref, a1c_ref, v0_ref,
             vx_ref, vy_ref, tq_ref):
        v0v = v0_ref[...]
        msum = a0m_ref[...] + a1m_ref[...]
        cnt = a0c_ref[...] + a1c_ref[...]
        tq_ref[...] = msum / jnp.maximum(cnt, 1.0)
        vx_ref[...] = v0v * c_ref[...]
        vy_ref[...] = v0v * s_ref[...]

    return pl.pallas_call(
        body,
        out_shape=(jax.ShapeDtypeStruct((rt, 128), F32),
                   jax.ShapeDtypeStruct((rt, 128), F32),
                   jax.ShapeDtypeStruct((rt, 128), F32)),
    )


def _sc_edge_stage(npad, n_chunks):
    q, r = divmod(n_chunks, NW)
    rows_per_tile = npad // NS
    mesh = plsc.VectorSubcoreMesh(core_axis_name="c", subcore_axis_name="s")

    @functools.partial(
        pl.kernel,
        out_type=jax.ShapeDtypeStruct((NC, npad, 2), F32),
        mesh=mesh,
        scratch_types=[
            pltpu.VMEM((CHROWS, 128), I32),
            pltpu.VMEM((CHROWS, 128), I32),
            pltpu.VMEM((CH, 5), F32),
            pltpu.VMEM((CH, 5), F32),
            pltpu.VMEM((CH, 2), F32),
            pltpu.VMEM((97 * 16,), F32),
            pltpu.VMEM_SHARED((npad, 2), F32),
            pltpu.SemaphoreType.DMA,
        ],
    )
    def sck(tab_hbm, src_hbm, dst_hbm, zeros_hbm, w_hbm, out_hbm,
            sidx, didx, srows, drows, mbuf, wv, acc, sem):
        cid = lax.axis_index("c")
        sid = lax.axis_index("s")
        wid = sid * NC + cid

        pltpu.sync_copy(w_hbm, wv)
        pltpu.sync_copy(zeros_hbm.at[pl.ds(sid * rows_per_tile, rows_per_tile)],
                        acc.at[pl.ds(sid * rows_per_tile, rows_per_tile)])

        iota = lax.iota(I32, 16)
        col0 = jnp.full((16,), 0, I32)
        col1 = jnp.full((16,), 1, I32)
        col2 = jnp.full((16,), 2, I32)
        col3 = jnp.full((16,), 3, I32)
        onesf = jnp.full((16,), 1.0, F32)

        def fill_ones(it, carry):
            plsc.store_scatter(mbuf, [it * 16 + iota, col1], onesf)
            return carry

        lax.fori_loop(0, CH // 16, fill_ones, None)
        plsc.subcore_barrier()

        niter = q + (wid < r).astype(I32)
        start_chunk = wid * q + jnp.minimum(wid, r)

        def wload(widx):
            return wv[pl.ds(widx * 16, 16)]

        def chunk_body(it, carry):
            crow = (start_chunk + it) * CHROWS
            pltpu.sync_copy(src_hbm.at[pl.ds(crow, CHROWS)], sidx)
            pltpu.sync_copy(dst_hbm.at[pl.ds(crow, CHROWS)], didx)
            descs = []
            for j in range(CHROWS):
                descs.append(pltpu.async_copy(
                    tab_hbm.at[sidx.at[j]], srows.at[pl.ds(j * 128, 128)], sem))
                descs.append(pltpu.async_copy(
                    tab_hbm.at[didx.at[j]], drows.at[pl.ds(j * 128, 128)], sem))
            for d in descs:
                d.wait()

            def block_body(b, bcarry):
                b1s = wload(96)
                feats = []
                maccs = []
                evs = []
                for kk in range(4):
                    e = (b * 4 + kk) * 16 + iota
                    evs.append(e)
                    xs0 = plsc.load_gather(srows, [e, col0])
                    xs1 = plsc.load_gather(srows, [e, col1])
                    cs = plsc.load_gather(srows, [e, col2])
                    ss = plsc.load_gather(srows, [e, col3])
                    xd0 = plsc.load_gather(drows, [e, col0])
                    xd1 = plsc.load_gather(drows, [e, col1])
                    cd = plsc.load_gather(drows, [e, col2])
                    sd = plsc.load_gather(drows, [e, col3])
                    dxw = xd0 - xs0
                    dyw = xd1 - xs1
                    dx = dxw * cs + dyw * ss
                    dy = dyw * cs - dxw * ss
                    cc = cd * cs + sd * ss
                    sn = sd * cs - cd * ss
                    feats.append((dx, dy, cc, sn))
                    maccs.append(b1s)
                for j in range(16):
                    w0a = wload(j)
                    w0b = wload(16 + j)
                    w0c = wload(32 + j)
                    w0d = wload(48 + j)
                    b0j = wload(64 + j)
                    w1j = wload(80 + j)
                    for kk in range(4):
                        dx, dy, cc, sn = feats[kk]
                        h = dx * w0a + dy * w0b + cc * w0c + sn * w0d + b0j
                        h = jnp.maximum(h, 0.0)
                        maccs[kk] = maccs[kk] + h * w1j
                for kk in range(4):
                    plsc.store_scatter(mbuf, [evs[kk], col0], maccs[kk])
                return bcarry

            lax.fori_loop(0, CH // 64, block_body, None)
            for j in range(CHROWS):
                pltpu.sync_copy(mbuf.at[pl.ds(j * 128, 128)],
                                acc.at[didx.at[j]], add=True)
            return carry

        lax.fori_loop(0, niter, chunk_body, None)
        plsc.subcore_barrier()
        pltpu.sync_copy(acc.at[pl.ds(sid * rows_per_tile, rows_per_tile)],
                        out_hbm.at[cid].at[pl.ds(sid * rows_per_tile,
                                                 rows_per_tile)])

    return sck


def kernel(x, theta, edge_index, W0, b0, W1, b1, v0):
    n = x.shape[0]
    e = edge_index.shape[1]
    npad = ((n + 127) // 128) * 128
    rt = npad // 128
    pad = npad - n

    thp = jnp.pad(theta[:, 0], (0, pad))
    c2, s2 = _node_stage(rt)(thp.reshape(rt, 128))
    cflat = c2.reshape(-1)
    sflat = s2.reshape(-1)
    xp0 = jnp.pad(x[:, 0], (0, pad))
    xp1 = jnp.pad(x[:, 1], (0, pad))
    table = jnp.stack([xp0, xp1, cflat, sflat, jnp.zeros_like(cflat)], axis=1)

    src2 = edge_index[0].reshape(e // 128, 128)
    dst2 = edge_index[1].reshape(e // 128, 128)

    wflat = jnp.concatenate([W0.reshape(-1), b0, W1.reshape(-1), b1])
    wvb = jnp.broadcast_to(wflat[:, None], (97, 16)).reshape(-1)
    zeros2 = jnp.zeros((npad, 2), F32)

    acc = _sc_edge_stage(npad, e // CH)(table, src2, dst2, zeros2, wvb)

    a0m = acc[0, :, 0].reshape(rt, 128)
    a1m = acc[1, :, 0].reshape(rt, 128)
    a0c = acc[0, :, 1].reshape(rt, 128)
    a1c = acc[1, :, 1].reshape(rt, 128)
    vx, vy, tq = _finalize_stage(rt)(
        c2, s2, a0m, a1m, a0c, a1c, v0.reshape(1, 1))
    out = jnp.stack([vx.reshape(-1), vy.reshape(-1), tq.reshape(-1)], axis=1)
    return out[:n]


# speedup vs baseline: 50.1939x; 50.1939x over previous
import functools

import jax
import jax.numpy as jnp
from jax import lax
from jax.experimental import pallas as pl
from jax.experimental.pallas import tpu as pltpu
from jax.experimental.pallas import tpu_sc as plsc

F32 = jnp.float32
I32 = jnp.int32

NS = 16
NC = 2
NW = NS * NC
CH = 1024
CHROWS = CH // 128
ROWW = 16


def _node_stage(rt):
    def body(th_ref, c_ref, s_ref):
        th = th_ref[...]
        c_ref[...] = jnp.cos(th)
        s_ref[...] = jnp.sin(th)

    return pl.pallas_call(
        body,
        out_shape=(jax.ShapeDtypeStruct((rt, 128), F32),
                   jax.ShapeDtypeStruct((rt, 128), F32)),
    )


def _finalize_stage(rt):
    def body(c_ref, s_ref, a0m_ref, a1m_ref, a0c_ref, a1c_ref, v0_ref,
             vx_ref, vy_ref, tq_ref):
        v0v = v0_ref[...]
        msum = a0m_ref[...] + a1m_ref[...]
        cnt = a0c_ref[...] + a1c_ref[...]
        tq_ref[...] = msum / jnp.maximum(cnt, 1.0)
        vx_ref[...] = v0v * c_ref[...]
        vy_ref[...] = v0v * s_ref[...]

    return pl.pallas_call(
        body,
        out_shape=(jax.ShapeDtypeStruct((rt, 128), F32),
                   jax.ShapeDtypeStruct((rt, 128), F32),
                   jax.ShapeDtypeStruct((rt, 128), F32)),
    )


def _sc_edge_stage(npad, n_chunks):
    q, r = divmod(n_chunks, NW)
    rows_per_tile = npad // NS
    mesh = plsc.VectorSubcoreMesh(core_axis_name="c", subcore_axis_name="s")

    @functools.partial(
        pl.kernel,
        out_type=(jax.ShapeDtypeStruct((NC, npad), F32),
                  jax.ShapeDtypeStruct((NC, npad), F32)),
        mesh=mesh,
        compiler_params=pltpu.CompilerParams(needs_layout_passes=False,
                                             use_tc_tiling_on_sc=False),
        scratch_types=[
            pltpu.VMEM((CHROWS, 128), I32),
            pltpu.VMEM((CHROWS, 128), I32),
            pltpu.VMEM((CH, ROWW), F32),
            pltpu.VMEM((CH, ROWW), F32),
            pltpu.VMEM((CH,), F32),
            pltpu.VMEM((128,), F32),
            pltpu.VMEM((97 * 16,), F32),
            pltpu.VMEM_SHARED((npad,), F32),
            pltpu.VMEM_SHARED((npad,), F32),
            pltpu.SemaphoreType.DMA,
        ],
    )
    def sck(tab_hbm, src_hbm, dst_hbm, zeros_hbm, w_hbm, outm_hbm, outc_hbm,
            sidx, didx, srows, drows, mbuf, onesb, wv, macc_sh, cacc_sh, sem):
        cid = lax.axis_index("c")
        sid = lax.axis_index("s")
        wid = sid * NC + cid

        pltpu.sync_copy(w_hbm, wv)
        pltpu.sync_copy(zeros_hbm.at[pl.ds(sid * rows_per_tile, rows_per_tile)],
                        macc_sh.at[pl.ds(sid * rows_per_tile, rows_per_tile)])
        pltpu.sync_copy(zeros_hbm.at[pl.ds(sid * rows_per_tile, rows_per_tile)],
                        cacc_sh.at[pl.ds(sid * rows_per_tile, rows_per_tile)])

        iota = lax.iota(I32, 16)
        col0 = jnp.full((16,), 0, I32)
        col1 = jnp.full((16,), 1, I32)
        col2 = jnp.full((16,), 2, I32)
        col3 = jnp.full((16,), 3, I32)
        onesf = jnp.full((16,), 1.0, F32)
        for i in range(128 // 16):
            onesb[pl.ds(i * 16, 16)] = onesf
        plsc.subcore_barrier()

        niter = q + (wid < r).astype(I32)
        start_chunk = wid * q + jnp.minimum(wid, r)

        def wload(widx):
            return wv[pl.ds(widx * 16, 16)]

        def chunk_body(it, carry):
            crow = (start_chunk + it) * CHROWS
            pltpu.sync_copy(src_hbm.at[pl.ds(crow, CHROWS)], sidx)
            pltpu.sync_copy(dst_hbm.at[pl.ds(crow, CHROWS)], didx)
            for j in range(CHROWS):
                pltpu.sync_copy(tab_hbm.at[sidx.at[j]],
                                srows.at[pl.ds(j * 128, 128)])
                pltpu.sync_copy(tab_hbm.at[didx.at[j]],
                                drows.at[pl.ds(j * 128, 128)])

            def block_body(b, bcarry):
                b1s = wload(96)
                feats = []
                maccs = []
                evs = []
                for kk in range(4):
                    e = (b * 4 + kk) * 16 + iota
                    evs.append(e)
                    xs0 = plsc.load_gather(srows, [e, col0])
                    xs1 = plsc.load_gather(srows, [e, col1])
                    cs = plsc.load_gather(srows, [e, col2])
                    ss = plsc.load_gather(srows, [e, col3])
                    xd0 = plsc.load_gather(drows, [e, col0])
                    xd1 = plsc.load_gather(drows, [e, col1])
                    cd = plsc.load_gather(drows, [e, col2])
                    sd = plsc.load_gather(drows, [e, col3])
                    dxw = xd0 - xs0
                    dyw = xd1 - xs1
                    dx = dxw * cs + dyw * ss
                    dy = dyw * cs - dxw * ss
                    cc = cd * cs + sd * ss
                    sn = sd * cs - cd * ss
                    feats.append((dx, dy, cc, sn))
                    maccs.append(b1s)
                for j in range(16):
                    w0a = wload(j)
                    w0b = wload(16 + j)
                    w0c = wload(32 + j)
                    w0d = wload(48 + j)
                    b0j = wload(64 + j)
                    w1j = wload(80 + j)
                    for kk in range(4):
                        dx, dy, cc, sn = feats[kk]
                        h = dx * w0a + dy * w0b + cc * w0c + sn * w0d + b0j
                        h = jnp.maximum(h, 0.0)
                        maccs[kk] = maccs[kk] + h * w1j
                for kk in range(4):
                    plsc.store_scatter(mbuf, [evs[kk]], maccs[kk])
                return bcarry

            lax.fori_loop(0, CH // 64, block_body, None)
            for j in range(CHROWS):
                pltpu.sync_copy(mbuf.at[pl.ds(j * 128, 128)],
                                macc_sh.at[didx.at[j]], add=True)
                pltpu.sync_copy(onesb, cacc_sh.at[didx.at[j]], add=True)
            return carry

        lax.fori_loop(0, niter, chunk_body, None)
        plsc.subcore_barrier()
        pltpu.sync_copy(macc_sh.at[pl.ds(sid * rows_per_tile, rows_per_tile)],
                        outm_hbm.at[cid].at[pl.ds(sid * rows_per_tile,
                                                  rows_per_tile)])
        pltpu.sync_copy(cacc_sh.at[pl.ds(sid * rows_per_tile, rows_per_tile)],
                        outc_hbm.at[cid].at[pl.ds(sid * rows_per_tile,
                                                  rows_per_tile)])

    return sck


def kernel(x, theta, edge_index, W0, b0, W1, b1, v0):
    n = x.shape[0]
    e = edge_index.shape[1]
    npad = ((n + 2047) // 2048) * 2048
    rt = npad // 128
    pad = npad - n

    thp = jnp.pad(theta[:, 0], (0, pad))
    c2, s2 = _node_stage(rt)(thp.reshape(rt, 128))
    cflat = c2.reshape(-1)
    sflat = s2.reshape(-1)
    xp0 = jnp.pad(x[:, 0], (0, pad))
    xp1 = jnp.pad(x[:, 1], (0, pad))
    zcol = jnp.zeros_like(cflat)
    table = jnp.stack([xp0, xp1, cflat, sflat] + [zcol] * (ROWW - 4), axis=1)

    src2 = edge_index[0].reshape(e // 128, 128)
    dst2 = edge_index[1].reshape(e // 128, 128)

    wflat = jnp.concatenate([W0.reshape(-1), b0, W1.reshape(-1), b1])
    wvb = jnp.broadcast_to(wflat[:, None], (97, 16)).reshape(-1)
    zeros1 = jnp.zeros((npad,), F32)

    accm, accc = _sc_edge_stage(npad, e // CH)(table, src2, dst2, zeros1, wvb)

    a0m = accm[0].reshape(rt, 128)
    a1m = accm[1].reshape(rt, 128)
    a0c = accc[0].reshape(rt, 128)
    a1c = accc[1].reshape(rt, 128)
    vx, vy, tq = _finalize_stage(rt)(
        c2, s2, a0m, a1m, a0c, a1c, v0.reshape(1, 1))
    out = jnp.stack([vx.reshape(-1), vy.reshape(-1), tq.reshape(-1)], axis=1)
    return out[:n]


# speedup vs baseline: 95.5070x; 1.9028x over previous
import functools

import jax
import jax.numpy as jnp
from jax import lax
from jax.experimental import pallas as pl
from jax.experimental.pallas import tpu as pltpu
from jax.experimental.pallas import tpu_sc as plsc

F32 = jnp.float32
I32 = jnp.int32

NS = 16
NC = 2
NW = NS * NC
CH = 1024
CHROWS = CH // 128
ROWW = 16


def _node_stage(rt):
    def body(th_ref, c_ref, s_ref):
        th = th_ref[...]
        c_ref[...] = jnp.cos(th)
        s_ref[...] = jnp.sin(th)

    return pl.pallas_call(
        body,
        out_shape=(jax.ShapeDtypeStruct((rt, 128), F32),
                   jax.ShapeDtypeStruct((rt, 128), F32)),
    )


def _finalize_stage(rt):
    def body(c_ref, s_ref, a0m_ref, a1m_ref, a0c_ref, a1c_ref, v0_ref,
             vx_ref, vy_ref, tq_ref):
        v0v = v0_ref[...]
        msum = a0m_ref[...] + a1m_ref[...]
        cnt = a0c_ref[...] + a1c_ref[...]
        tq_ref[...] = msum / jnp.maximum(cnt, 1.0)
        vx_ref[...] = v0v * c_ref[...]
        vy_ref[...] = v0v * s_ref[...]

    return pl.pallas_call(
        body,
        out_shape=(jax.ShapeDtypeStruct((rt, 128), F32),
                   jax.ShapeDtypeStruct((rt, 128), F32),
                   jax.ShapeDtypeStruct((rt, 128), F32)),
    )


def _sc_edge_stage(npad, n_chunks):
    q, r = divmod(n_chunks, NW)
    rows_per_tile = npad // NS
    mesh = plsc.VectorSubcoreMesh(core_axis_name="c", subcore_axis_name="s")

    @functools.partial(
        pl.kernel,
        out_type=(jax.ShapeDtypeStruct((NC, npad), F32),
                  jax.ShapeDtypeStruct((NC, npad), F32)),
        mesh=mesh,
        compiler_params=pltpu.CompilerParams(needs_layout_passes=False,
                                             use_tc_tiling_on_sc=False),
        scratch_types=[
            pltpu.VMEM((CHROWS, 128), I32),
            pltpu.VMEM((CHROWS, 128), I32),
            pltpu.VMEM((CH, ROWW), F32),
            pltpu.VMEM((CH, ROWW), F32),
            pltpu.VMEM((CH,), F32),
            pltpu.VMEM((128,), F32),
            pltpu.VMEM((97 * 16,), F32),
            pltpu.VMEM_SHARED((npad,), F32),
            pltpu.VMEM_SHARED((npad,), F32),
            pltpu.SemaphoreType.DMA,
        ],
    )
    def sck(tab_hbm, src_hbm, dst_hbm, zeros_hbm, w_hbm, outm_hbm, outc_hbm,
            sidx, didx, srows, drows, mbuf, onesb, wv, macc_sh, cacc_sh, sem):
        cid = lax.axis_index("c")
        sid = lax.axis_index("s")
        wid = sid * NC + cid

        pltpu.sync_copy(w_hbm, wv)
        pltpu.sync_copy(zeros_hbm.at[pl.ds(sid * rows_per_tile, rows_per_tile)],
                        macc_sh.at[pl.ds(sid * rows_per_tile, rows_per_tile)])
        pltpu.sync_copy(zeros_hbm.at[pl.ds(sid * rows_per_tile, rows_per_tile)],
                        cacc_sh.at[pl.ds(sid * rows_per_tile, rows_per_tile)])

        iota = lax.iota(I32, 16)
        col0 = jnp.full((16,), 0, I32)
        col1 = jnp.full((16,), 1, I32)
        col2 = jnp.full((16,), 2, I32)
        col3 = jnp.full((16,), 3, I32)
        onesf = jnp.full((16,), 1.0, F32)
        for i in range(128 // 16):
            onesb[pl.ds(i * 16, 16)] = onesf
        plsc.subcore_barrier()

        niter = q + (wid < r).astype(I32)
        start_chunk = wid * q + jnp.minimum(wid, r)

        def wload(widx):
            return wv[pl.ds(widx * 16, 16)]

        def chunk_body(it, carry):
            crow = (start_chunk + it) * CHROWS
            pltpu.sync_copy(src_hbm.at[pl.ds(crow, CHROWS)], sidx)
            pltpu.sync_copy(dst_hbm.at[pl.ds(crow, CHROWS)], didx)
            descs = []
            for j in range(CHROWS):
                descs.append(pltpu.async_copy(
                    tab_hbm.at[sidx.at[j]], srows.at[pl.ds(j * 128, 128)],
                    sem))
                descs.append(pltpu.async_copy(
                    tab_hbm.at[didx.at[j]], drows.at[pl.ds(j * 128, 128)],
                    sem))
            for d in descs:
                d.wait()

            def block_body(b, bcarry):
                b1s = wload(96)
                feats = []
                maccs = []
                evs = []
                for kk in range(4):
                    e = (b * 4 + kk) * 16 + iota
                    evs.append(e)
                    xs0 = plsc.load_gather(srows, [e, col0])
                    xs1 = plsc.load_gather(srows, [e, col1])
                    cs = plsc.load_gather(srows, [e, col2])
                    ss = plsc.load_gather(srows, [e, col3])
                    xd0 = plsc.load_gather(drows, [e, col0])
                    xd1 = plsc.load_gather(drows, [e, col1])
                    cd = plsc.load_gather(drows, [e, col2])
                    sd = plsc.load_gather(drows, [e, col3])
                    dxw = xd0 - xs0
                    dyw = xd1 - xs1
                    dx = dxw * cs + dyw * ss
                    dy = dyw * cs - dxw * ss
                    cc = cd * cs + sd * ss
                    sn = sd * cs - cd * ss
                    feats.append((dx, dy, cc, sn))
                    maccs.append(b1s)
                for j in range(16):
                    w0a = wload(j)
                    w0b = wload(16 + j)
                    w0c = wload(32 + j)
                    w0d = wload(48 + j)
                    b0j = wload(64 + j)
                    w1j = wload(80 + j)
                    for kk in range(4):
                        dx, dy, cc, sn = feats[kk]
                        h = dx * w0a + dy * w0b + cc * w0c + sn * w0d + b0j
                        h = jnp.maximum(h, 0.0)
                        maccs[kk] = maccs[kk] + h * w1j
                for kk in range(4):
                    plsc.store_scatter(mbuf, [evs[kk]], maccs[kk])
                return bcarry

            lax.fori_loop(0, CH // 64, block_body, None)
            for j in range(CHROWS):
                pltpu.sync_copy(mbuf.at[pl.ds(j * 128, 128)],
                                macc_sh.at[didx.at[j]], add=True)
                pltpu.sync_copy(onesb, cacc_sh.at[didx.at[j]], add=True)
            return carry

        lax.fori_loop(0, niter, chunk_body, None)
        plsc.subcore_barrier()
        pltpu.sync_copy(macc_sh.at[pl.ds(sid * rows_per_tile, rows_per_tile)],
                        outm_hbm.at[cid].at[pl.ds(sid * rows_per_tile,
                                                  rows_per_tile)])
        pltpu.sync_copy(cacc_sh.at[pl.ds(sid * rows_per_tile, rows_per_tile)],
                        outc_hbm.at[cid].at[pl.ds(sid * rows_per_tile,
                                                  rows_per_tile)])

    return sck


def kernel(x, theta, edge_index, W0, b0, W1, b1, v0):
    n = x.shape[0]
    e = edge_index.shape[1]
    npad = ((n + 2047) // 2048) * 2048
    rt = npad // 128
    pad = npad - n

    thp = jnp.pad(theta[:, 0], (0, pad))
    c2, s2 = _node_stage(rt)(thp.reshape(rt, 128))
    cflat = c2.reshape(-1)
    sflat = s2.reshape(-1)
    xp0 = jnp.pad(x[:, 0], (0, pad))
    xp1 = jnp.pad(x[:, 1], (0, pad))
    zcol = jnp.zeros_like(cflat)
    table = jnp.stack([xp0, xp1, cflat, sflat] + [zcol] * (ROWW - 4), axis=1)

    src2 = edge_index[0].reshape(e // 128, 128)
    dst2 = edge_index[1].reshape(e // 128, 128)

    wflat = jnp.concatenate([W0.reshape(-1), b0, W1.reshape(-1), b1])
    wvb = jnp.broadcast_to(wflat[:, None], (97, 16)).reshape(-1)
    zeros1 = jnp.zeros((npad,), F32)

    accm, accc = _sc_edge_stage(npad, e // CH)(table, src2, dst2, zeros1, wvb)

    a0m = accm[0].reshape(rt, 128)
    a1m = accm[1].reshape(rt, 128)
    a0c = accc[0].reshape(rt, 128)
    a1c = accc[1].reshape(rt, 128)
    vx, vy, tq = _finalize_stage(rt)(
        c2, s2, a0m, a1m, a0c, a1c, v0.reshape(1, 1))
    out = jnp.stack([vx.reshape(-1), vy.reshape(-1), tq.reshape(-1)], axis=1)
    return out[:n]
